# baseline (speedup 1.0000x reference)
import jax
import jax.numpy as jnp
from jax.experimental import pallas as pl
from jax.experimental.pallas import tpu as pltpu


def _clam_block(h_ref, W1_ref, b1_ref, Wa_ref, ba_ref, Wb_ref, bb_ref,
                wc_ref, bc_ref, Wcls_ref, bcls_ref,
                A_ref, logits_ref, yprob_ref, yhat_ref,
                m_ref, s_ref, acc_ref):
    i = pl.program_id(0)
    nblk = pl.num_programs(0)

    @pl.when(i == 0)
    def _init():
        m_ref[...] = jnp.full_like(m_ref, -jnp.inf)
        s_ref[...] = jnp.zeros_like(s_ref)
        acc_ref[...] = jnp.zeros_like(acc_ref)

    h_blk = h_ref[...].astype(jnp.bfloat16)
    h1 = jnp.dot(h_blk, W1_ref[...], preferred_element_type=jnp.float32)
    h1 = jnp.maximum(h1 + b1_ref[...], 0.0)
    h1b = h1.astype(jnp.bfloat16)
    a = jnp.tanh(jnp.dot(h1b, Wa_ref[...], preferred_element_type=jnp.float32)
                 + ba_ref[...])
    b = jax.nn.sigmoid(jnp.dot(h1b, Wb_ref[...], preferred_element_type=jnp.float32)
                       + bb_ref[...])
    g = a * b
    att = jnp.sum(g * wc_ref[...], axis=1, keepdims=True) + bc_ref[...]
    A_ref[...] = att

    m_prev = m_ref[...]
    m_new = jnp.maximum(m_prev, jnp.max(att, axis=(0, 1), keepdims=True))
    p = jnp.exp(att - m_new)
    corr = jnp.exp(m_prev - m_new)
    s_ref[...] = s_ref[...] * corr + jnp.sum(p, axis=(0, 1), keepdims=True)
    pw = jax.lax.dot_general(p, h1, (((0,), (0,)), ((), ())),
                             preferred_element_type=jnp.float32)
    acc_ref[...] = acc_ref[...] * corr + pw
    m_ref[...] = m_new

    @pl.when(i == nblk - 1)
    def _finish():
        M = acc_ref[...] / s_ref[...]
        logits = jnp.dot(M, Wcls_ref[...], preferred_element_type=jnp.float32)
        logits = logits + bcls_ref[...]
        logits_ref[...] = logits
        mx = jnp.max(logits, axis=1, keepdims=True)
        e = jnp.exp(logits - mx)
        yprob_ref[...] = e / jnp.sum(e, axis=1, keepdims=True)
        c = logits.shape[1]
        idx = jax.lax.broadcasted_iota(jnp.int32, logits.shape, 1)
        yhat_ref[...] = jnp.min(jnp.where(logits == mx, idx, c), axis=1,
                                keepdims=True)


def kernel(h, W1, b1, Wa, ba, Wb, bb, Wc, bc, Wcls, bcls):
    n, d_in = h.shape
    d_hid = W1.shape[1]
    d_att = Wa.shape[1]
    n_classes = Wcls.shape[1]

    blk = 1000
    if n % blk != 0:
        blk = next(b for b in (500, 250, 200, 100, 50, 25, 10, 8, 5, 4, 2, 1)
                   if n % b == 0)
    nblk = n // blk

    W1_b = W1.astype(jnp.bfloat16)
    Wa_b = Wa.astype(jnp.bfloat16)
    Wb_b = Wb.astype(jnp.bfloat16)
    b1_r = b1.reshape(1, d_hid)
    ba_r = ba.reshape(1, d_att)
    bb_r = bb.reshape(1, d_att)
    wc_r = Wc.reshape(1, d_att)
    bc_r = bc.reshape(1, 1)
    bcls_r = bcls.reshape(1, n_classes)

    const = lambda i: (0, 0)
    out = pl.pallas_call(
        _clam_block,
        grid=(nblk,),
        in_specs=[
            pl.BlockSpec((blk, d_in), lambda i: (i, 0)),
            pl.BlockSpec((d_in, d_hid), const),
            pl.BlockSpec((1, d_hid), const),
            pl.BlockSpec((d_hid, d_att), const),
            pl.BlockSpec((1, d_att), const),
            pl.BlockSpec((d_hid, d_att), const),
            pl.BlockSpec((1, d_att), const),
            pl.BlockSpec((1, d_att), const),
            pl.BlockSpec((1, 1), const),
            pl.BlockSpec((d_hid, n_classes), const),
            pl.BlockSpec((1, n_classes), const),
        ],
        out_specs=[
            pl.BlockSpec((blk, 1), lambda i: (i, 0)),
            pl.BlockSpec((1, n_classes), const),
            pl.BlockSpec((1, n_classes), const),
            pl.BlockSpec((1, 1), const),
        ],
        out_shape=[
            jax.ShapeDtypeStruct((n, 1), jnp.float32),
            jax.ShapeDtypeStruct((1, n_classes), jnp.float32),
            jax.ShapeDtypeStruct((1, n_classes), jnp.float32),
            jax.ShapeDtypeStruct((1, 1), jnp.int32),
        ],
        scratch_shapes=[
            pltpu.VMEM((1, 1), jnp.float32),
            pltpu.VMEM((1, 1), jnp.float32),
            pltpu.VMEM((1, d_hid), jnp.float32),
        ],
        compiler_params=pltpu.CompilerParams(
            dimension_semantics=("arbitrary",),
        ),
    )(h, W1_b, b1_r, Wa_b, ba_r, Wb_b, bb_r, wc_r, bc_r, Wcls, bcls_r)

    A_col, logits, y_prob, y_hat = out
    return (logits, y_prob, y_hat, A_col.reshape(1, n))


# speedup vs baseline: 1.0329x; 1.0329x over previous
import jax
import jax.numpy as jnp
from jax.experimental import pallas as pl
from jax.experimental.pallas import tpu as pltpu


def _clam_block(h_ref, W1_ref, b1_ref, Wa_ref, ba_ref, Wb_ref, bb_ref,
                wc_ref, bc_ref, Wcls_ref, bcls_ref,
                A_ref, logits_ref, yprob_ref, yhat_ref,
                m_ref, s_ref, acc_ref):
    i = pl.program_id(0)
    nblk = pl.num_programs(0)

    @pl.when(i == 0)
    def _init():
        m_ref[...] = (jnp.sum(jnp.abs(wc_ref[...]), axis=1, keepdims=True)
                      + jnp.abs(bc_ref[...]))
        s_ref[...] = jnp.zeros_like(s_ref)
        acc_ref[...] = jnp.zeros_like(acc_ref)

    h_blk = h_ref[...].astype(jnp.bfloat16)
    h1 = jnp.dot(h_blk, W1_ref[...], preferred_element_type=jnp.float32)
    h1 = jnp.maximum(h1 + b1_ref[...], 0.0)
    h1b = h1.astype(jnp.bfloat16)
    a = jnp.tanh(jnp.dot(h1b, Wa_ref[...], preferred_element_type=jnp.float32)
                 + ba_ref[...])
    b = jax.nn.sigmoid(jnp.dot(h1b, Wb_ref[...], preferred_element_type=jnp.float32)
                       + bb_ref[...])
    g = a * b
    att = jnp.sum(g * wc_ref[...], axis=1, keepdims=True) + bc_ref[...]
    A_ref[...] = att

    p = jnp.exp(att - m_ref[...])
    s_ref[...] = s_ref[...] + jnp.sum(p, axis=(0, 1), keepdims=True)
    pw = jax.lax.dot_general(p, h1, (((0,), (0,)), ((), ())),
                             preferred_element_type=jnp.float32)
    acc_ref[...] = acc_ref[...] + pw

    @pl.when(i == nblk - 1)
    def _finish():
        M = acc_ref[...] / s_ref[...]
        logits = jnp.dot(M, Wcls_ref[...], preferred_element_type=jnp.float32)
        logits = logits + bcls_ref[...]
        logits_ref[...] = logits
        mx = jnp.max(logits, axis=1, keepdims=True)
        e = jnp.exp(logits - mx)
        yprob_ref[...] = e / jnp.sum(e, axis=1, keepdims=True)
        c = logits.shape[1]
        idx = jax.lax.broadcasted_iota(jnp.int32, logits.shape, 1)
        yhat_ref[...] = jnp.min(jnp.where(logits == mx, idx, c), axis=1,
                                keepdims=True)


def kernel(h, W1, b1, Wa, ba, Wb, bb, Wc, bc, Wcls, bcls):
    n, d_in = h.shape
    d_hid = W1.shape[1]
    d_att = Wa.shape[1]
    n_classes = Wcls.shape[1]

    blk = 2000
    if n % blk != 0:
        blk = next(b for b in (500, 250, 200, 100, 50, 25, 10, 8, 5, 4, 2, 1)
                   if n % b == 0)
    nblk = n // blk

    W1_b = W1.astype(jnp.bfloat16)
    Wa_b = Wa.astype(jnp.bfloat16)
    Wb_b = Wb.astype(jnp.bfloat16)
    b1_r = b1.reshape(1, d_hid)
    ba_r = ba.reshape(1, d_att)
    bb_r = bb.reshape(1, d_att)
    wc_r = Wc.reshape(1, d_att)
    bc_r = bc.reshape(1, 1)
    bcls_r = bcls.reshape(1, n_classes)

    const = lambda i: (0, 0)
    out = pl.pallas_call(
        _clam_block,
        grid=(nblk,),
        in_specs=[
            pl.BlockSpec((blk, d_in), lambda i: (i, 0)),
            pl.BlockSpec((d_in, d_hid), const),
            pl.BlockSpec((1, d_hid), const),
            pl.BlockSpec((d_hid, d_att), const),
            pl.BlockSpec((1, d_att), const),
            pl.BlockSpec((d_hid, d_att), const),
            pl.BlockSpec((1, d_att), const),
            pl.BlockSpec((1, d_att), const),
            pl.BlockSpec((1, 1), const),
            pl.BlockSpec((d_hid, n_classes), const),
            pl.BlockSpec((1, n_classes), const),
        ],
        out_specs=[
            pl.BlockSpec((blk, 1), lambda i: (i, 0)),
            pl.BlockSpec((1, n_classes), const),
            pl.BlockSpec((1, n_classes), const),
            pl.BlockSpec((1, 1), const),
        ],
        out_shape=[
            jax.ShapeDtypeStruct((n, 1), jnp.float32),
            jax.ShapeDtypeStruct((1, n_classes), jnp.float32),
            jax.ShapeDtypeStruct((1, n_classes), jnp.float32),
            jax.ShapeDtypeStruct((1, 1), jnp.int32),
        ],
        scratch_shapes=[
            pltpu.VMEM((1, 1), jnp.float32),
            pltpu.VMEM((1, 1), jnp.float32),
            pltpu.VMEM((1, d_hid), jnp.float32),
        ],
        compiler_params=pltpu.CompilerParams(
            dimension_semantics=("arbitrary",),
        ),
    )(h, W1_b, b1_r, Wa_b, ba_r, Wb_b, bb_r, wc_r, bc_r, Wcls, bcls_r)

    A_col, logits, y_prob, y_hat = out
    return (logits, y_prob, y_hat, A_col.reshape(1, n))
